# baseline (speedup 1.0000x reference)
import functools

import jax
import jax.numpy as jnp
from jax import lax
from jax.experimental import pallas as pl
from jax.experimental.pallas import tpu as pltpu
from jax.experimental.pallas import tpu_sc as plsc

B = 16384
F = 26
V = 2600000
E = 16
H1 = 512
H2 = 256
EPS = 1e-5

BF = B * F
NC, NS = 2, 16
NW = NC * NS
PER_W = BF // NW
CHUNK = 128
NBUF = 4
NGRP = PER_W // (CHUNK * NBUF)

RB = 1024


def _sc_gather_body(idx_hbm, t2_hbm, t1_hbm, rows_out, vals_out,
                    idx_v, rows_v, vals_v, gsem, vsem, osem):
    wid = lax.axis_index("s") * NC + lax.axis_index("c")
    base = wid * PER_W
    pltpu.sync_copy(idx_hbm.at[pl.ds(base, PER_W)], idx_v)

    def group(g, carry):
        j0 = g * (CHUNK * NBUF)
        for b in range(NBUF):
            off = j0 + b * CHUNK
            idx_c = idx_v.at[pl.ds(off, CHUNK)]
            pltpu.async_copy(t2_hbm.at[idx_c], rows_v.at[pl.ds(b * CHUNK, CHUNK)], gsem)
            pltpu.async_copy(t1_hbm.at[idx_c], vals_v.at[pl.ds(off, CHUNK)], vsem)
        for b in range(NBUF):
            off = j0 + b * CHUNK
            pltpu.make_async_copy(t2_hbm.at[idx_v.at[pl.ds(off, CHUNK)]],
                                  rows_v.at[pl.ds(b * CHUNK, CHUNK)], gsem).wait()
            pltpu.async_copy(rows_v.at[pl.ds(b * CHUNK, CHUNK)],
                             rows_out.at[pl.ds(base + off, CHUNK)], osem)
        for b in range(NBUF):
            off = j0 + b * CHUNK
            pltpu.make_async_copy(t1_hbm.at[idx_v.at[pl.ds(off, CHUNK)]],
                                  vals_v.at[pl.ds(off, CHUNK)], vsem).wait()
        for b in range(NBUF):
            off = j0 + b * CHUNK
            pltpu.make_async_copy(rows_v.at[pl.ds(b * CHUNK, CHUNK)],
                                  rows_out.at[pl.ds(base + off, CHUNK)], osem).wait()
        return carry

    lax.fori_loop(0, NGRP, group, 0)
    pltpu.sync_copy(vals_v, vals_out.at[pl.ds(base, PER_W)])


@functools.lru_cache(maxsize=1)
def _sc_gather():
    return pl.kernel(
        _sc_gather_body,
        out_type=[jax.ShapeDtypeStruct((BF, E), jnp.float32),
                  jax.ShapeDtypeStruct((BF,), jnp.float32)],
        mesh=plsc.VectorSubcoreMesh(core_axis_name="c", subcore_axis_name="s",
                                    num_cores=NC, num_subcores=NS),
        scratch_types=[pltpu.VMEM((PER_W,), jnp.int32),
                       pltpu.VMEM((NBUF * CHUNK, E), jnp.float32),
                       pltpu.VMEM((PER_W,), jnp.float32),
                       pltpu.SemaphoreType.DMA,
                       pltpu.SemaphoreType.DMA,
                       pltpu.SemaphoreType.DMA],
        compiler_params=pltpu.CompilerParams(use_tc_tiling_on_sc=False),
    )


def _tc_body(d_ref, em_ref, w1_ref, b1_ref, w2_ref, b2_ref, s_ref, o_ref):
    d = d_ref[...]
    smat = s_ref[...]
    es = jnp.dot(d, smat, preferred_element_type=jnp.float32)
    esq = jnp.dot(d * d, smat, preferred_element_type=jnp.float32)
    fm = 0.5 * jnp.sum(es * es - esq, axis=1)
    first = jnp.sum(em_ref[...], axis=1)
    h = jnp.dot(d, w1_ref[...], preferred_element_type=jnp.float32) + b1_ref[...]
    h = jnp.maximum(h, 0.0)
    h = jnp.dot(h, w2_ref[...], preferred_element_type=jnp.float32) + b2_ref[...]
    h = jnp.maximum(h, 0.0)
    total = first + fm + jnp.sum(h, axis=1)
    o_ref[...] = jax.nn.sigmoid(total)


def kernel(X, emb1_w, emb2_w, W1, b1, gamma1, beta1, W2, b2, gamma2, beta2):
    Xi = X.reshape(BF).astype(jnp.int32)
    rows, vals = _sc_gather()(Xi, emb2_w, emb1_w.reshape(V))
    d = rows.reshape(B, F * E)
    em = vals.reshape(B, F)

    inv = 1.0 / jnp.sqrt(1.0 + EPS)
    g1 = gamma1 * inv
    w1f = W1 * g1[None, :]
    b1f = (b1 * g1 + beta1)[None, :]
    g2 = gamma2 * inv
    w2f = W2 * g2[None, :]
    b2f = (b2 * g2 + beta2)[None, :]
    smat = jnp.tile(jnp.eye(E, dtype=jnp.float32), (F, 1))

    out = pl.pallas_call(
        _tc_body,
        grid=(B // RB,),
        in_specs=[
            pl.BlockSpec((RB, F * E), lambda i: (i, 0)),
            pl.BlockSpec((RB, F), lambda i: (i, 0)),
            pl.BlockSpec((F * E, H1), lambda i: (0, 0)),
            pl.BlockSpec((1, H1), lambda i: (0, 0)),
            pl.BlockSpec((H1, H2), lambda i: (0, 0)),
            pl.BlockSpec((1, H2), lambda i: (0, 0)),
            pl.BlockSpec((F * E, E), lambda i: (0, 0)),
        ],
        out_specs=pl.BlockSpec((RB,), lambda i: (i,)),
        out_shape=jax.ShapeDtypeStruct((B,), jnp.float32),
    )(d, em, w1f, b1f, w2f, b2f, smat)
    return out


# speedup vs baseline: 1.5330x; 1.5330x over previous
import functools

import jax
import jax.numpy as jnp
from jax import lax
from jax.experimental import pallas as pl
from jax.experimental.pallas import tpu as pltpu
from jax.experimental.pallas import tpu_sc as plsc

B = 16384
F = 26
V = 2600000
E = 16
H1 = 512
H2 = 256
EPS = 1e-5

BF = B * F
NC, NS = 2, 16
NW = NC * NS
PER_W = BF // NW
CHUNK = 128
NCHUNK = PER_W // CHUNK

VB = 65536
RB = 1024


def _detile_body(in_ref, *o_refs):
    x = in_ref[...]
    for e in range(E):
        o_refs[e][...] = x[e]


def _detile(t):
    import math
    grid = (math.ceil(V / VB),)
    return pl.pallas_call(
        _detile_body,
        grid=grid,
        in_specs=[pl.BlockSpec((E, VB), lambda i: (0, i))],
        out_specs=[pl.BlockSpec((VB,), lambda i: (i,)) for _ in range(E)],
        out_shape=[jax.ShapeDtypeStruct((V,), jnp.float32) for _ in range(E)],
    )(t)


def _sc_gather_body(idx_hbm, *rest):
    tabs = rest[:E]
    t1_hbm = rest[E]
    rows_out, vals_out = rest[E + 1], rest[E + 2]
    idx_v, gbuf, rbuf, vals_v, gsem, vsem = rest[E + 3:]

    wid = lax.axis_index("s") * NC + lax.axis_index("c")
    base = wid * PER_W
    pltpu.sync_copy(idx_hbm.at[pl.ds(base, PER_W)], idx_v)
    lanes = lax.iota(jnp.int32, 16)

    def chunk(j, carry):
        off = j * CHUNK
        idx_c = idx_v.at[pl.ds(off, CHUNK)]
        for e in range(E):
            pltpu.async_copy(tabs[e].at[idx_c], gbuf.at[e], gsem)
        pltpu.async_copy(t1_hbm.at[idx_c], vals_v.at[pl.ds(off, CHUNK)], vsem)
        for e in range(E):
            pltpu.make_async_copy(tabs[e].at[idx_c], gbuf.at[e], gsem).wait()

        def row8(i0, c2):
            for di in range(8):
                i = i0 * 8 + di
                v = plsc.load_gather(gbuf, [lanes, jnp.full((16,), i, jnp.int32)])
                rbuf[i, :] = v
            return c2

        lax.fori_loop(0, CHUNK // 8, row8, 0)
        pltpu.make_async_copy(t1_hbm.at[idx_c],
                              vals_v.at[pl.ds(off, CHUNK)], vsem).wait()
        pltpu.sync_copy(rbuf, rows_out.at[pl.ds(base + off, CHUNK)])
        return carry

    lax.fori_loop(0, NCHUNK, chunk, 0)
    pltpu.sync_copy(vals_v, vals_out.at[pl.ds(base, PER_W)])


@functools.lru_cache(maxsize=1)
def _sc_gather():
    return pl.kernel(
        _sc_gather_body,
        out_type=[jax.ShapeDtypeStruct((BF, E), jnp.float32),
                  jax.ShapeDtypeStruct((BF,), jnp.float32)],
        mesh=plsc.VectorSubcoreMesh(core_axis_name="c", subcore_axis_name="s",
                                    num_cores=NC, num_subcores=NS),
        scratch_types=[pltpu.VMEM((PER_W,), jnp.int32),
                       pltpu.VMEM((E, CHUNK), jnp.float32),
                       pltpu.VMEM((CHUNK, E), jnp.float32),
                       pltpu.VMEM((PER_W,), jnp.float32),
                       pltpu.SemaphoreType.DMA,
                       pltpu.SemaphoreType.DMA],
        compiler_params=pltpu.CompilerParams(use_tc_tiling_on_sc=False,
                                             needs_layout_passes=False),
    )


def _tc_body(d_ref, em_ref, w1_ref, b1_ref, w2_ref, b2_ref, s_ref, o_ref):
    d = d_ref[...]
    smat = s_ref[...]
    es = jnp.dot(d, smat, preferred_element_type=jnp.float32)
    esq = jnp.dot(d * d, smat, preferred_element_type=jnp.float32)
    fm = 0.5 * jnp.sum(es * es - esq, axis=1)
    first = jnp.sum(em_ref[...], axis=1)
    h = jnp.dot(d, w1_ref[...], preferred_element_type=jnp.float32) + b1_ref[...]
    h = jnp.maximum(h, 0.0)
    h = jnp.dot(h, w2_ref[...], preferred_element_type=jnp.float32) + b2_ref[...]
    h = jnp.maximum(h, 0.0)
    total = first + fm + jnp.sum(h, axis=1)
    o_ref[...] = jax.nn.sigmoid(total)


def kernel(X, emb1_w, emb2_w, W1, b1, gamma1, beta1, W2, b2, gamma2, beta2):
    Xi = X.reshape(BF).astype(jnp.int32)
    tabs = _detile(emb2_w.T)
    rows, vals = _sc_gather()(Xi, *tabs, emb1_w.reshape(V))
    d = rows.reshape(B, F * E)
    em = vals.reshape(B, F)

    inv = 1.0 / jnp.sqrt(1.0 + EPS)
    g1 = gamma1 * inv
    w1f = W1 * g1[None, :]
    b1f = (b1 * g1 + beta1)[None, :]
    g2 = gamma2 * inv
    w2f = W2 * g2[None, :]
    b2f = (b2 * g2 + beta2)[None, :]
    smat = jnp.tile(jnp.eye(E, dtype=jnp.float32), (F, 1))

    out = pl.pallas_call(
        _tc_body,
        grid=(B // RB,),
        in_specs=[
            pl.BlockSpec((RB, F * E), lambda i: (i, 0)),
            pl.BlockSpec((RB, F), lambda i: (i, 0)),
            pl.BlockSpec((F * E, H1), lambda i: (0, 0)),
            pl.BlockSpec((1, H1), lambda i: (0, 0)),
            pl.BlockSpec((H1, H2), lambda i: (0, 0)),
            pl.BlockSpec((1, H2), lambda i: (0, 0)),
            pl.BlockSpec((F * E, E), lambda i: (0, 0)),
        ],
        out_specs=pl.BlockSpec((RB,), lambda i: (i,)),
        out_shape=jax.ShapeDtypeStruct((B,), jnp.float32),
    )(d, em, w1f, b1f, w2f, b2f, smat)
    return out


# speedup vs baseline: 2.0873x; 1.3616x over previous
import functools

import jax
import jax.numpy as jnp
from jax import lax
from jax.experimental import pallas as pl
from jax.experimental.pallas import tpu as pltpu
from jax.experimental.pallas import tpu_sc as plsc

B = 16384
F = 26
V = 2600000
E = 16
H1 = 512
H2 = 256
EPS = 1e-5

BF = B * F
NC, NS = 2, 16
NW = NC * NS
PER_W = BF // NW
CHUNK = 128
NCHUNK = PER_W // CHUNK

VB = 65536
RB = 1024


def _detile_body(in_ref, in1_ref, *o_refs):
    x = in_ref[...]
    for e in range(E):
        o_refs[e][...] = x[e]
    o_refs[E][...] = in1_ref[...][0]


def _detile(t, t1):
    import math
    grid = (math.ceil(V / VB),)
    return pl.pallas_call(
        _detile_body,
        grid=grid,
        in_specs=[pl.BlockSpec((E, VB), lambda i: (0, i)),
                  pl.BlockSpec((1, VB), lambda i: (0, i))],
        out_specs=[pl.BlockSpec((VB,), lambda i: (i,)) for _ in range(E + 1)],
        out_shape=[jax.ShapeDtypeStruct((V,), jnp.float32) for _ in range(E + 1)],
    )(t, t1)


def _sc_gather_body(idx_hbm, *rest):
    tabs = rest[:E]
    t1_hbm = rest[E]
    rows_out, vals_out = rest[E + 1], rest[E + 2]
    idx_v, gbuf, rbuf, vals_v, gsem, vsem = rest[E + 3:]

    wid = lax.axis_index("s") * NC + lax.axis_index("c")
    base = wid * PER_W
    pltpu.sync_copy(idx_hbm.at[pl.ds(base, PER_W)], idx_v)
    lanes = lax.iota(jnp.int32, 16)

    def fire(j, s):
        idx_c = idx_v.at[pl.ds(j * CHUNK, CHUNK)]
        for e in range(E):
            pltpu.async_copy(tabs[e].at[idx_c], gbuf.at[s, e], gsem)
        pltpu.async_copy(t1_hbm.at[idx_c],
                         vals_v.at[pl.ds(j * CHUNK, CHUNK)], vsem)

    def drain_interleave_out(j, s):
        idx_c = idx_v.at[pl.ds(j * CHUNK, CHUNK)]
        for e in range(E):
            pltpu.make_async_copy(tabs[e].at[idx_c], gbuf.at[s, e], gsem).wait()

        def row8(i0, c2):
            for di in range(8):
                i = i0 * 8 + di
                v = plsc.load_gather(gbuf.at[s],
                                     [lanes, jnp.full((16,), i, jnp.int32)])
                rbuf[i, :] = v
            return c2

        lax.fori_loop(0, CHUNK // 8, row8, 0)
        pltpu.make_async_copy(t1_hbm.at[idx_c],
                              vals_v.at[pl.ds(j * CHUNK, CHUNK)], vsem).wait()
        pltpu.sync_copy(rbuf, rows_out.at[pl.ds(base + j * CHUNK, CHUNK)])

    fire(0, 0)

    def pair(jj, carry):
        for s in range(2):
            j = jj * 2 + s
            jn = j + 1

            @pl.when(jn < NCHUNK)
            def _():
                fire(jn, 1 - s)

            drain_interleave_out(j, s)
        return carry

    lax.fori_loop(0, NCHUNK // 2, pair, 0)
    pltpu.sync_copy(vals_v, vals_out.at[pl.ds(base, PER_W)])


@functools.lru_cache(maxsize=1)
def _sc_gather():
    return pl.kernel(
        _sc_gather_body,
        out_type=[jax.ShapeDtypeStruct((BF, E), jnp.float32),
                  jax.ShapeDtypeStruct((BF,), jnp.float32)],
        mesh=plsc.VectorSubcoreMesh(core_axis_name="c", subcore_axis_name="s",
                                    num_cores=NC, num_subcores=NS),
        scratch_types=[pltpu.VMEM((PER_W,), jnp.int32),
                       pltpu.VMEM((2, E, CHUNK), jnp.float32),
                       pltpu.VMEM((CHUNK, E), jnp.float32),
                       pltpu.VMEM((PER_W,), jnp.float32),
                       pltpu.SemaphoreType.DMA,
                       pltpu.SemaphoreType.DMA],
        compiler_params=pltpu.CompilerParams(use_tc_tiling_on_sc=False,
                                             needs_layout_passes=False),
    )


def _tc_body(d_ref, em_ref, w1_ref, b1_ref, w2_ref, b2_ref, s_ref, o_ref):
    d = d_ref[...]
    smat = s_ref[...]
    es = jnp.dot(d, smat, preferred_element_type=jnp.float32)
    esq = jnp.dot(d * d, smat, preferred_element_type=jnp.float32)
    fm = 0.5 * jnp.sum(es * es - esq, axis=1)
    first = jnp.sum(em_ref[...], axis=1)
    h = jnp.dot(d, w1_ref[...], preferred_element_type=jnp.float32) + b1_ref[...]
    h = jnp.maximum(h, 0.0)
    h = jnp.dot(h, w2_ref[...], preferred_element_type=jnp.float32) + b2_ref[...]
    h = jnp.maximum(h, 0.0)
    total = first + fm + jnp.sum(h, axis=1)
    o_ref[...] = jax.nn.sigmoid(total)


def kernel(X, emb1_w, emb2_w, W1, b1, gamma1, beta1, W2, b2, gamma2, beta2):
    Xi = X.reshape(BF).astype(jnp.int32)
    tabs = _detile(emb2_w.T, emb1_w.T)
    rows, vals = _sc_gather()(Xi, *tabs)
    d = rows.reshape(B, F * E)
    em = vals.reshape(B, F)

    inv = 1.0 / jnp.sqrt(1.0 + EPS)
    g1 = gamma1 * inv
    w1f = W1 * g1[None, :]
    b1f = (b1 * g1 + beta1)[None, :]
    g2 = gamma2 * inv
    w2f = W2 * g2[None, :]
    b2f = (b2 * g2 + beta2)[None, :]
    smat = jnp.tile(jnp.eye(E, dtype=jnp.float32), (F, 1))

    out = pl.pallas_call(
        _tc_body,
        grid=(B // RB,),
        in_specs=[
            pl.BlockSpec((RB, F * E), lambda i: (i, 0)),
            pl.BlockSpec((RB, F), lambda i: (i, 0)),
            pl.BlockSpec((F * E, H1), lambda i: (0, 0)),
            pl.BlockSpec((1, H1), lambda i: (0, 0)),
            pl.BlockSpec((H1, H2), lambda i: (0, 0)),
            pl.BlockSpec((1, H2), lambda i: (0, 0)),
            pl.BlockSpec((F * E, E), lambda i: (0, 0)),
        ],
        out_specs=pl.BlockSpec((RB,), lambda i: (i,)),
        out_shape=jax.ShapeDtypeStruct((B,), jnp.float32),
    )(d, em, w1f, b1f, w2f, b2f, smat)
    return out


# speedup vs baseline: 2.3041x; 1.1038x over previous
import functools

import jax
import jax.numpy as jnp
from jax import lax
from jax.experimental import pallas as pl
from jax.experimental.pallas import tpu as pltpu
from jax.experimental.pallas import tpu_sc as plsc

B = 16384
F = 26
V = 2600000
E = 16
H1 = 512
H2 = 256
EPS = 1e-5

BF = B * F
NC, NS = 2, 16
NW = NC * NS
PER_W = BF // NW
CHUNK = 128
NCHUNK = PER_W // CHUNK

VB = 65536
RB = 1024


def _detile_body(in_ref, in1_ref, *o_refs):
    x = in_ref[...]
    for e in range(E):
        o_refs[e][...] = x[e]
    o_refs[E][...] = in1_ref[...][0]


def _detile(t, t1):
    import math
    grid = (math.ceil(V / VB),)
    return pl.pallas_call(
        _detile_body,
        grid=grid,
        in_specs=[pl.BlockSpec((E, VB), lambda i: (0, i)),
                  pl.BlockSpec((1, VB), lambda i: (0, i))],
        out_specs=[pl.BlockSpec((VB,), lambda i: (i,)) for _ in range(E + 1)],
        out_shape=[jax.ShapeDtypeStruct((V,), jnp.float32) for _ in range(E + 1)],
    )(t, t1)


def _sc_gather_body(idx_hbm, *rest):
    tabs = rest[:E]
    t1_hbm = rest[E]
    rows_out, vals_out = rest[E + 1], rest[E + 2]
    idx_v, gbuf, rbuf, vals_v, gsem, vsem = rest[E + 3:]

    wid = lax.axis_index("s") * NC + lax.axis_index("c")
    base = wid * PER_W
    pltpu.sync_copy(idx_hbm.at[pl.ds(base, PER_W)], idx_v)
    lanes = lax.iota(jnp.int32, 16)

    def fire(j, s):
        idx_c = idx_v.at[pl.ds(j * CHUNK, CHUNK)]
        for e in range(E):
            pltpu.async_copy(tabs[e].at[idx_c], gbuf.at[s, e], gsem)
        pltpu.async_copy(t1_hbm.at[idx_c],
                         vals_v.at[pl.ds(j * CHUNK, CHUNK)], vsem)

    def drain_interleave_out(j, s):
        idx_c = idx_v.at[pl.ds(j * CHUNK, CHUNK)]
        for e in range(E):
            pltpu.make_async_copy(tabs[e].at[idx_c], gbuf.at[s, e], gsem).wait()

        def row8(i0, c2):
            for di in range(8):
                i = i0 * 8 + di
                v = plsc.load_gather(gbuf.at[s],
                                     [lanes, jnp.full((16,), i, jnp.int32)])
                rbuf[i, :] = v
            return c2

        lax.fori_loop(0, CHUNK // 8, row8, 0)
        pltpu.make_async_copy(t1_hbm.at[idx_c],
                              vals_v.at[pl.ds(j * CHUNK, CHUNK)], vsem).wait()
        pltpu.sync_copy(rbuf, rows_out.at[pl.ds(base + j * CHUNK, CHUNK)])

    for p in range(3):
        fire(p, p)

    def quad(jj, carry):
        for s in range(4):
            j = jj * 4 + s
            jn = j + 3

            @pl.when(jn < NCHUNK)
            def _():
                fire(jn, (s + 3) % 4)

            drain_interleave_out(j, s)
        return carry

    lax.fori_loop(0, NCHUNK // 4, quad, 0)
    pltpu.sync_copy(vals_v, vals_out.at[pl.ds(base, PER_W)])


@functools.lru_cache(maxsize=1)
def _sc_gather():
    return pl.kernel(
        _sc_gather_body,
        out_type=[jax.ShapeDtypeStruct((BF, E), jnp.float32),
                  jax.ShapeDtypeStruct((BF,), jnp.float32)],
        mesh=plsc.VectorSubcoreMesh(core_axis_name="c", subcore_axis_name="s",
                                    num_cores=NC, num_subcores=NS),
        scratch_types=[pltpu.VMEM((PER_W,), jnp.int32),
                       pltpu.VMEM((4, E, CHUNK), jnp.float32),
                       pltpu.VMEM((CHUNK, E), jnp.float32),
                       pltpu.VMEM((PER_W,), jnp.float32),
                       pltpu.SemaphoreType.DMA,
                       pltpu.SemaphoreType.DMA],
        compiler_params=pltpu.CompilerParams(use_tc_tiling_on_sc=False,
                                             needs_layout_passes=False),
    )


def _tc_body(d_ref, em_ref, w1_ref, b1_ref, w2_ref, b2_ref, s_ref, o_ref):
    d = d_ref[...]
    smat = s_ref[...]
    es = jnp.dot(d, smat, preferred_element_type=jnp.float32)
    esq = jnp.dot(d * d, smat, preferred_element_type=jnp.float32)
    fm = 0.5 * jnp.sum(es * es - esq, axis=1)
    first = jnp.sum(em_ref[...], axis=1)
    h = jnp.dot(d.astype(jnp.bfloat16), w1_ref[...],
                preferred_element_type=jnp.float32) + b1_ref[...]
    h = jnp.maximum(h, 0.0)
    h = jnp.dot(h.astype(jnp.bfloat16), w2_ref[...],
                preferred_element_type=jnp.float32) + b2_ref[...]
    h = jnp.maximum(h, 0.0)
    total = first + fm + jnp.sum(h, axis=1)
    o_ref[...] = jax.nn.sigmoid(total)


def kernel(X, emb1_w, emb2_w, W1, b1, gamma1, beta1, W2, b2, gamma2, beta2):
    Xi = X.reshape(BF).astype(jnp.int32)
    tabs = _detile(emb2_w.T, emb1_w.T)
    rows, vals = _sc_gather()(Xi, *tabs)
    d = rows.reshape(B, F * E)
    em = vals.reshape(B, F)

    inv = 1.0 / jnp.sqrt(1.0 + EPS)
    g1 = gamma1 * inv
    w1f = (W1 * g1[None, :]).astype(jnp.bfloat16)
    b1f = (b1 * g1 + beta1)[None, :]
    g2 = gamma2 * inv
    w2f = (W2 * g2[None, :]).astype(jnp.bfloat16)
    b2f = (b2 * g2 + beta2)[None, :]
    smat = jnp.tile(jnp.eye(E, dtype=jnp.float32), (F, 1))

    out = pl.pallas_call(
        _tc_body,
        grid=(B // RB,),
        in_specs=[
            pl.BlockSpec((RB, F * E), lambda i: (i, 0)),
            pl.BlockSpec((RB, F), lambda i: (i, 0)),
            pl.BlockSpec((F * E, H1), lambda i: (0, 0)),
            pl.BlockSpec((1, H1), lambda i: (0, 0)),
            pl.BlockSpec((H1, H2), lambda i: (0, 0)),
            pl.BlockSpec((1, H2), lambda i: (0, 0)),
            pl.BlockSpec((F * E, E), lambda i: (0, 0)),
        ],
        out_specs=pl.BlockSpec((RB,), lambda i: (i,)),
        out_shape=jax.ShapeDtypeStruct((B,), jnp.float32),
    )(d, em, w1f, b1f, w2f, b2f, smat)
    return out


# speedup vs baseline: 2.3166x; 1.0054x over previous
import functools

import jax
import jax.numpy as jnp
from jax import lax
from jax.experimental import pallas as pl
from jax.experimental.pallas import tpu as pltpu
from jax.experimental.pallas import tpu_sc as plsc

B = 16384
F = 26
V = 2600000
E = 16
H1 = 512
H2 = 256
EPS = 1e-5

BF = B * F
NC, NS = 2, 16
NW = NC * NS
PER_W = BF // NW
CHUNK = 128
NCHUNK = PER_W // CHUNK

VB = 65536
RB = 1024


def _detile_body(in_ref, in1_ref, *o_refs):
    x = in_ref[...]
    for e in range(E):
        o_refs[e][...] = x[e]
    o_refs[E][...] = in1_ref[...][0]


def _detile(t, t1):
    import math
    grid = (math.ceil(V / VB),)
    return pl.pallas_call(
        _detile_body,
        grid=grid,
        in_specs=[pl.BlockSpec((E, VB), lambda i: (0, i)),
                  pl.BlockSpec((1, VB), lambda i: (0, i))],
        out_specs=[pl.BlockSpec((VB,), lambda i: (i,)) for _ in range(E + 1)],
        out_shape=[jax.ShapeDtypeStruct((V,), jnp.float32) for _ in range(E + 1)],
    )(t, t1)


def _sc_gather_body(idx_hbm, *rest):
    tabs = rest[:E]
    t1_hbm = rest[E]
    rows_out, vals_out = rest[E + 1], rest[E + 2]
    idx_v, gbuf, rbuf, vals_v, gsem, vsem, osem = rest[E + 3:]

    wid = lax.axis_index("s") * NC + lax.axis_index("c")
    base = wid * PER_W
    pltpu.sync_copy(idx_hbm.at[pl.ds(base, PER_W)], idx_v)
    lanes = lax.iota(jnp.int32, 16)

    def fire(j, s):
        idx_c = idx_v.at[pl.ds(j * CHUNK, CHUNK)]
        for e in range(E):
            pltpu.async_copy(tabs[e].at[idx_c], gbuf.at[s, e], gsem)
        pltpu.async_copy(t1_hbm.at[idx_c],
                         vals_v.at[pl.ds(j * CHUNK, CHUNK)], vsem)

    def out_desc(j, s):
        return pltpu.make_async_copy(
            rbuf.at[s], rows_out.at[pl.ds(base + j * CHUNK, CHUNK)], osem)

    def drain_interleave_out(j, s):
        idx_c = idx_v.at[pl.ds(j * CHUNK, CHUNK)]
        for e in range(E):
            pltpu.make_async_copy(tabs[e].at[idx_c], gbuf.at[s, e], gsem).wait()

        @pl.when(j >= 4)
        def _():
            out_desc(j - 4, s).wait()

        def row8(i0, c2):
            for di in range(8):
                i = i0 * 8 + di
                v = plsc.load_gather(gbuf.at[s],
                                     [lanes, jnp.full((16,), i, jnp.int32)])
                rbuf[s, i, :] = v
            return c2

        lax.fori_loop(0, CHUNK // 8, row8, 0)
        pltpu.make_async_copy(t1_hbm.at[idx_c],
                              vals_v.at[pl.ds(j * CHUNK, CHUNK)], vsem).wait()
        pltpu.async_copy(rbuf.at[s],
                         rows_out.at[pl.ds(base + j * CHUNK, CHUNK)], osem)

    for p in range(3):
        fire(p, p)

    def quad(jj, carry):
        for s in range(4):
            j = jj * 4 + s
            jn = j + 3

            @pl.when(jn < NCHUNK)
            def _():
                fire(jn, (s + 3) % 4)

            drain_interleave_out(j, s)
        return carry

    lax.fori_loop(0, NCHUNK // 4, quad, 0)
    for s in range(4):
        out_desc(NCHUNK - 4 + s, s).wait()
    pltpu.sync_copy(vals_v, vals_out.at[pl.ds(base, PER_W)])


@functools.lru_cache(maxsize=1)
def _sc_gather():
    return pl.kernel(
        _sc_gather_body,
        out_type=[jax.ShapeDtypeStruct((BF, E), jnp.float32),
                  jax.ShapeDtypeStruct((BF,), jnp.float32)],
        mesh=plsc.VectorSubcoreMesh(core_axis_name="c", subcore_axis_name="s",
                                    num_cores=NC, num_subcores=NS),
        scratch_types=[pltpu.VMEM((PER_W,), jnp.int32),
                       pltpu.VMEM((4, E, CHUNK), jnp.float32),
                       pltpu.VMEM((4, CHUNK, E), jnp.float32),
                       pltpu.VMEM((PER_W,), jnp.float32),
                       pltpu.SemaphoreType.DMA,
                       pltpu.SemaphoreType.DMA,
                       pltpu.SemaphoreType.DMA],
        compiler_params=pltpu.CompilerParams(use_tc_tiling_on_sc=False,
                                             needs_layout_passes=False),
    )


def _tc_body(d_ref, em_ref, w1_ref, b1_ref, w2_ref, b2_ref, s_ref, o_ref):
    d = d_ref[...]
    smat = s_ref[...]
    es = jnp.dot(d, smat, preferred_element_type=jnp.float32)
    esq = jnp.dot(d * d, smat, preferred_element_type=jnp.float32)
    fm = 0.5 * jnp.sum(es * es - esq, axis=1)
    first = jnp.sum(em_ref[...], axis=1)
    h = jnp.dot(d.astype(jnp.bfloat16), w1_ref[...],
                preferred_element_type=jnp.float32) + b1_ref[...]
    h = jnp.maximum(h, 0.0)
    h = jnp.dot(h.astype(jnp.bfloat16), w2_ref[...],
                preferred_element_type=jnp.float32) + b2_ref[...]
    h = jnp.maximum(h, 0.0)
    total = first + fm + jnp.sum(h, axis=1)
    o_ref[...] = jax.nn.sigmoid(total)


def kernel(X, emb1_w, emb2_w, W1, b1, gamma1, beta1, W2, b2, gamma2, beta2):
    Xi = X.reshape(BF).astype(jnp.int32)
    tabs = _detile(emb2_w.T, emb1_w.T)
    rows, vals = _sc_gather()(Xi, *tabs)
    d = rows.reshape(B, F * E)
    em = vals.reshape(B, F)

    inv = 1.0 / jnp.sqrt(1.0 + EPS)
    g1 = gamma1 * inv
    w1f = (W1 * g1[None, :]).astype(jnp.bfloat16)
    b1f = (b1 * g1 + beta1)[None, :]
    g2 = gamma2 * inv
    w2f = (W2 * g2[None, :]).astype(jnp.bfloat16)
    b2f = (b2 * g2 + beta2)[None, :]
    smat = jnp.tile(jnp.eye(E, dtype=jnp.float32), (F, 1))

    out = pl.pallas_call(
        _tc_body,
        grid=(B // RB,),
        in_specs=[
            pl.BlockSpec((RB, F * E), lambda i: (i, 0)),
            pl.BlockSpec((RB, F), lambda i: (i, 0)),
            pl.BlockSpec((F * E, H1), lambda i: (0, 0)),
            pl.BlockSpec((1, H1), lambda i: (0, 0)),
            pl.BlockSpec((H1, H2), lambda i: (0, 0)),
            pl.BlockSpec((1, H2), lambda i: (0, 0)),
            pl.BlockSpec((F * E, E), lambda i: (0, 0)),
        ],
        out_specs=pl.BlockSpec((RB,), lambda i: (i,)),
        out_shape=jax.ShapeDtypeStruct((B,), jnp.float32),
    )(d, em, w1f, b1f, w2f, b2f, smat)
    return out


# speedup vs baseline: 2.3981x; 1.0352x over previous
import functools

import jax
import jax.numpy as jnp
from jax import lax
from jax.experimental import pallas as pl
from jax.experimental.pallas import tpu as pltpu
from jax.experimental.pallas import tpu_sc as plsc

B = 16384
F = 26
V = 2600000
E = 16
H1 = 512
H2 = 256
EPS = 1e-5

BF = B * F
NC, NS = 2, 16
NW = NC * NS
PER_W = BF // NW
CHUNK = 128
NCHUNK = PER_W // CHUNK
NSLOT = 8

VB = 65536
RB = 1024


def _detile_body(in_ref, in1_ref, *o_refs):
    x = in_ref[...]
    for e in range(E):
        o_refs[e][...] = x[e]
    o_refs[E][...] = in1_ref[...][0]


def _detile(t, t1):
    import math
    grid = (math.ceil(V / VB),)
    return pl.pallas_call(
        _detile_body,
        grid=grid,
        in_specs=[pl.BlockSpec((E, VB), lambda i: (0, i)),
                  pl.BlockSpec((1, VB), lambda i: (0, i))],
        out_specs=[pl.BlockSpec((VB,), lambda i: (i,)) for _ in range(E + 1)],
        out_shape=[jax.ShapeDtypeStruct((V,), jnp.float32) for _ in range(E + 1)],
    )(t, t1)


def _sc_gather_body(idx_hbm, *rest):
    tabs = rest[:E]
    t1_hbm = rest[E]
    rows_out, vals_out = rest[E + 1], rest[E + 2]
    idx_v, gbuf, rbuf, vals_v, gsem, vsem, osem = rest[E + 3:]

    wid = lax.axis_index("s") * NC + lax.axis_index("c")
    base = wid * PER_W
    pltpu.sync_copy(idx_hbm.at[pl.ds(base, PER_W)], idx_v)
    lanes = lax.iota(jnp.int32, 16)

    def fire(j, s):
        idx_c = idx_v.at[pl.ds(j * CHUNK, CHUNK)]
        for e in range(E):
            pltpu.async_copy(tabs[e].at[idx_c], gbuf.at[s, e], gsem)
        pltpu.async_copy(t1_hbm.at[idx_c],
                         vals_v.at[pl.ds(j * CHUNK, CHUNK)], vsem)

    def out_desc(j, s):
        return pltpu.make_async_copy(
            rbuf.at[s], rows_out.at[pl.ds(base + j * CHUNK, CHUNK)], osem)

    def drain_interleave_out(j, s):
        idx_c = idx_v.at[pl.ds(j * CHUNK, CHUNK)]
        for e in range(E):
            pltpu.make_async_copy(tabs[e].at[idx_c], gbuf.at[s, e], gsem).wait()

        @pl.when(j >= NSLOT)
        def _():
            out_desc(j - NSLOT, s).wait()

        def row8(i0, c2):
            for di in range(8):
                i = i0 * 8 + di
                v = plsc.load_gather(gbuf.at[s],
                                     [lanes, jnp.full((16,), i, jnp.int32)])
                rbuf[s, i, :] = v
            return c2

        lax.fori_loop(0, CHUNK // 8, row8, 0)
        pltpu.make_async_copy(t1_hbm.at[idx_c],
                              vals_v.at[pl.ds(j * CHUNK, CHUNK)], vsem).wait()
        pltpu.async_copy(rbuf.at[s],
                         rows_out.at[pl.ds(base + j * CHUNK, CHUNK)], osem)

    for p in range(NSLOT - 1):
        fire(p, p)

    def ring(jj, carry):
        for s in range(NSLOT):
            j = jj * NSLOT + s
            jn = j + NSLOT - 1

            @pl.when(jn < NCHUNK)
            def _():
                fire(jn, (s + NSLOT - 1) % NSLOT)

            drain_interleave_out(j, s)
        return carry

    lax.fori_loop(0, NCHUNK // NSLOT, ring, 0)
    for s in range(NSLOT):
        out_desc(NCHUNK - NSLOT + s, s).wait()
    pltpu.sync_copy(vals_v, vals_out.at[pl.ds(base, PER_W)])


@functools.lru_cache(maxsize=1)
def _sc_gather():
    return pl.kernel(
        _sc_gather_body,
        out_type=[jax.ShapeDtypeStruct((BF, E), jnp.float32),
                  jax.ShapeDtypeStruct((BF,), jnp.float32)],
        mesh=plsc.VectorSubcoreMesh(core_axis_name="c", subcore_axis_name="s",
                                    num_cores=NC, num_subcores=NS),
        scratch_types=[pltpu.VMEM((PER_W,), jnp.int32),
                       pltpu.VMEM((NSLOT, E, CHUNK), jnp.float32),
                       pltpu.VMEM((NSLOT, CHUNK, E), jnp.float32),
                       pltpu.VMEM((PER_W,), jnp.float32),
                       pltpu.SemaphoreType.DMA,
                       pltpu.SemaphoreType.DMA,
                       pltpu.SemaphoreType.DMA],
        compiler_params=pltpu.CompilerParams(use_tc_tiling_on_sc=False,
                                             needs_layout_passes=False),
    )


def _tc_body(d_ref, em_ref, w1_ref, b1_ref, w2_ref, b2_ref, s_ref, o_ref):
    d = d_ref[...]
    smat = s_ref[...]
    es = jnp.dot(d, smat, preferred_element_type=jnp.float32)
    esq = jnp.dot(d * d, smat, preferred_element_type=jnp.float32)
    fm = 0.5 * jnp.sum(es * es - esq, axis=1)
    first = jnp.sum(em_ref[...], axis=1)
    h = jnp.dot(d.astype(jnp.bfloat16), w1_ref[...],
                preferred_element_type=jnp.float32) + b1_ref[...]
    h = jnp.maximum(h, 0.0)
    h = jnp.dot(h.astype(jnp.bfloat16), w2_ref[...],
                preferred_element_type=jnp.float32) + b2_ref[...]
    h = jnp.maximum(h, 0.0)
    total = first + fm + jnp.sum(h, axis=1)
    o_ref[...] = jax.nn.sigmoid(total)


def kernel(X, emb1_w, emb2_w, W1, b1, gamma1, beta1, W2, b2, gamma2, beta2):
    Xi = X.reshape(BF).astype(jnp.int32)
    tabs = _detile(emb2_w.T, emb1_w.T)
    rows, vals = _sc_gather()(Xi, *tabs)
    d = rows.reshape(B, F * E)
    em = vals.reshape(B, F)

    inv = 1.0 / jnp.sqrt(1.0 + EPS)
    g1 = gamma1 * inv
    w1f = (W1 * g1[None, :]).astype(jnp.bfloat16)
    b1f = (b1 * g1 + beta1)[None, :]
    g2 = gamma2 * inv
    w2f = (W2 * g2[None, :]).astype(jnp.bfloat16)
    b2f = (b2 * g2 + beta2)[None, :]
    smat = jnp.tile(jnp.eye(E, dtype=jnp.float32), (F, 1))

    out = pl.pallas_call(
        _tc_body,
        grid=(B // RB,),
        in_specs=[
            pl.BlockSpec((RB, F * E), lambda i: (i, 0)),
            pl.BlockSpec((RB, F), lambda i: (i, 0)),
            pl.BlockSpec((F * E, H1), lambda i: (0, 0)),
            pl.BlockSpec((1, H1), lambda i: (0, 0)),
            pl.BlockSpec((H1, H2), lambda i: (0, 0)),
            pl.BlockSpec((1, H2), lambda i: (0, 0)),
            pl.BlockSpec((F * E, E), lambda i: (0, 0)),
        ],
        out_specs=pl.BlockSpec((RB,), lambda i: (i,)),
        out_shape=jax.ShapeDtypeStruct((B,), jnp.float32),
    )(d, em, w1f, b1f, w2f, b2f, smat)
    return out


# speedup vs baseline: 2.4083x; 1.0042x over previous
import functools

import jax
import jax.numpy as jnp
from jax import lax
from jax.experimental import pallas as pl
from jax.experimental.pallas import tpu as pltpu
from jax.experimental.pallas import tpu_sc as plsc

B = 16384
F = 26
V = 2600000
E = 16
H1 = 512
H2 = 256
EPS = 1e-5

BF = B * F
NC, NS = 2, 16
NW = NC * NS
PER_W = BF // NW
CHUNK = 128
NCHUNK = PER_W // CHUNK
NSLOT = 13

VB = 65536
RB = 1024


def _detile_body(in_ref, in1_ref, *o_refs):
    x = in_ref[...]
    for e in range(E):
        o_refs[e][...] = x[e]
    o_refs[E][...] = in1_ref[...][0]


def _detile(t, t1):
    import math
    grid = (math.ceil(V / VB),)
    return pl.pallas_call(
        _detile_body,
        grid=grid,
        in_specs=[pl.BlockSpec((E, VB), lambda i: (0, i)),
                  pl.BlockSpec((1, VB), lambda i: (0, i))],
        out_specs=[pl.BlockSpec((VB,), lambda i: (i,)) for _ in range(E + 1)],
        out_shape=[jax.ShapeDtypeStruct((V,), jnp.float32) for _ in range(E + 1)],
    )(t, t1)


def _sc_gather_body(idx_hbm, *rest):
    tabs = rest[:E]
    t1_hbm = rest[E]
    rows_out, vals_out = rest[E + 1], rest[E + 2]
    idx_v, gbuf, rbuf, vals_v, gsem, vsem, osem = rest[E + 3:]

    wid = lax.axis_index("s") * NC + lax.axis_index("c")
    base = wid * PER_W
    pltpu.sync_copy(idx_hbm.at[pl.ds(base, PER_W)], idx_v)
    lanes = lax.iota(jnp.int32, 16)

    def fire(j, s):
        idx_c = idx_v.at[pl.ds(j * CHUNK, CHUNK)]
        for e in range(E):
            pltpu.async_copy(tabs[e].at[idx_c], gbuf.at[s, e], gsem)
        pltpu.async_copy(t1_hbm.at[idx_c],
                         vals_v.at[pl.ds(j * CHUNK, CHUNK)], vsem)

    def out_desc(j, s):
        return pltpu.make_async_copy(
            rbuf.at[s], rows_out.at[pl.ds(base + j * CHUNK, CHUNK)], osem)

    def drain_interleave_out(j, s):
        idx_c = idx_v.at[pl.ds(j * CHUNK, CHUNK)]
        for e in range(E):
            pltpu.make_async_copy(tabs[e].at[idx_c], gbuf.at[s, e], gsem).wait()

        @pl.when(j >= NSLOT)
        def _():
            out_desc(j - NSLOT, s).wait()

        def row8(i0, c2):
            for di in range(8):
                i = i0 * 8 + di
                v = plsc.load_gather(gbuf.at[s],
                                     [lanes, jnp.full((16,), i, jnp.int32)])
                rbuf[s, i, :] = v
            return c2

        lax.fori_loop(0, CHUNK // 8, row8, 0)
        pltpu.make_async_copy(t1_hbm.at[idx_c],
                              vals_v.at[pl.ds(j * CHUNK, CHUNK)], vsem).wait()
        pltpu.async_copy(rbuf.at[s],
                         rows_out.at[pl.ds(base + j * CHUNK, CHUNK)], osem)

    for p in range(NSLOT - 1):
        fire(p, p)

    def ring(jj, carry):
        for s in range(NSLOT):
            j = jj * NSLOT + s
            jn = j + NSLOT - 1

            @pl.when(jn < NCHUNK)
            def _():
                fire(jn, (s + NSLOT - 1) % NSLOT)

            drain_interleave_out(j, s)
        return carry

    lax.fori_loop(0, NCHUNK // NSLOT, ring, 0)
    for s in range(NSLOT):
        out_desc(NCHUNK - NSLOT + s, s).wait()
    pltpu.sync_copy(vals_v, vals_out.at[pl.ds(base, PER_W)])


@functools.lru_cache(maxsize=1)
def _sc_gather():
    return pl.kernel(
        _sc_gather_body,
        out_type=[jax.ShapeDtypeStruct((BF, E), jnp.float32),
                  jax.ShapeDtypeStruct((BF,), jnp.float32)],
        mesh=plsc.VectorSubcoreMesh(core_axis_name="c", subcore_axis_name="s",
                                    num_cores=NC, num_subcores=NS),
        scratch_types=[pltpu.VMEM((PER_W,), jnp.int32),
                       pltpu.VMEM((NSLOT, E, CHUNK), jnp.float32),
                       pltpu.VMEM((NSLOT, CHUNK, E), jnp.float32),
                       pltpu.VMEM((PER_W,), jnp.float32),
                       pltpu.SemaphoreType.DMA,
                       pltpu.SemaphoreType.DMA,
                       pltpu.SemaphoreType.DMA],
        compiler_params=pltpu.CompilerParams(use_tc_tiling_on_sc=False,
                                             needs_layout_passes=False),
    )


def _tc_body(d_ref, em_ref, w1_ref, b1_ref, w2_ref, b2_ref, s_ref, o_ref):
    d = d_ref[...]
    smat = s_ref[...]
    es = jnp.dot(d, smat, preferred_element_type=jnp.float32)
    esq = jnp.dot(d * d, smat, preferred_element_type=jnp.float32)
    fm = 0.5 * jnp.sum(es * es - esq, axis=1)
    first = jnp.sum(em_ref[...], axis=1)
    h = jnp.dot(d.astype(jnp.bfloat16), w1_ref[...],
                preferred_element_type=jnp.float32) + b1_ref[...]
    h = jnp.maximum(h, 0.0)
    h = jnp.dot(h.astype(jnp.bfloat16), w2_ref[...],
                preferred_element_type=jnp.float32) + b2_ref[...]
    h = jnp.maximum(h, 0.0)
    total = first + fm + jnp.sum(h, axis=1)
    o_ref[...] = jax.nn.sigmoid(total)


def kernel(X, emb1_w, emb2_w, W1, b1, gamma1, beta1, W2, b2, gamma2, beta2):
    Xi = X.reshape(BF).astype(jnp.int32)
    tabs = _detile(emb2_w.T, emb1_w.T)
    rows, vals = _sc_gather()(Xi, *tabs)
    d = rows.reshape(B, F * E)
    em = vals.reshape(B, F)

    inv = 1.0 / jnp.sqrt(1.0 + EPS)
    g1 = gamma1 * inv
    w1f = (W1 * g1[None, :]).astype(jnp.bfloat16)
    b1f = (b1 * g1 + beta1)[None, :]
    g2 = gamma2 * inv
    w2f = (W2 * g2[None, :]).astype(jnp.bfloat16)
    b2f = (b2 * g2 + beta2)[None, :]
    smat = jnp.tile(jnp.eye(E, dtype=jnp.float32), (F, 1))

    out = pl.pallas_call(
        _tc_body,
        grid=(B // RB,),
        in_specs=[
            pl.BlockSpec((RB, F * E), lambda i: (i, 0)),
            pl.BlockSpec((RB, F), lambda i: (i, 0)),
            pl.BlockSpec((F * E, H1), lambda i: (0, 0)),
            pl.BlockSpec((1, H1), lambda i: (0, 0)),
            pl.BlockSpec((H1, H2), lambda i: (0, 0)),
            pl.BlockSpec((1, H2), lambda i: (0, 0)),
            pl.BlockSpec((F * E, E), lambda i: (0, 0)),
        ],
        out_specs=pl.BlockSpec((RB,), lambda i: (i,)),
        out_shape=jax.ShapeDtypeStruct((B,), jnp.float32),
    )(d, em, w1f, b1f, w2f, b2f, smat)
    return out
